# baseline (speedup 1.0000x reference)
import jax
import jax.numpy as jnp
from jax import lax
from jax.experimental import pallas as pl
from jax.experimental.pallas import tpu as pltpu
from jax.experimental.pallas import tpu_sc as plsc

DIM = 128
SPAN = 10
LANES = 16
NUM_WORKERS = 32
CHUNK_B = 8
K_FIRE = 4
MACRO_B = CHUNK_B * K_FIRE


def _cbow_body(words_hbm, table_hbm, out_hbm, idx_v, rows0, rows1, out0,
               out1, sem0, sem1, sem_o0, sem_o1):
    b_per_w = out_hbm.shape[0] // NUM_WORKERS
    n_macro = b_per_w // MACRO_B
    wid = lax.axis_index("s") * 2 + lax.axis_index("c")
    n_idx = b_per_w * SPAN
    pltpu.sync_copy(words_hbm.at[pl.ds(wid * n_idx, n_idx)], idx_v)

    def fire(s, buf, sem):
        for k in range(K_FIRE):
            pltpu.async_copy(
                table_hbm.at[
                    idx_v.at[pl.ds(s * MACRO_B * SPAN + k * CHUNK_B * SPAN,
                                   CHUNK_B * SPAN)]
                ],
                buf.at[pl.ds(k * CHUNK_B * SPAN, CHUNK_B * SPAN)],
                sem,
            )

    def drain(buf, sem):
        for k in range(K_FIRE):
            pltpu.make_async_copy(
                table_hbm.at[idx_v.at[pl.ds(0, CHUNK_B * SPAN)]],
                buf.at[pl.ds(k * CHUNK_B * SPAN, CHUNK_B * SPAN)],
                sem,
            ).wait()

    def out_slice(s):
        return out_hbm.at[pl.ds(wid * b_per_w + s * MACRO_B, MACRO_B)]

    def compute(s, buf, out_v, sem_o):
        inv_span = jnp.float32(1.0 / SPAN)

        @plsc.parallel_loop(0, MACRO_B, unroll=4)
        def _(b):
            base = b * SPAN
            for g in range(DIM // LANES):
                sl = pl.ds(g * LANES, LANES)
                acc = buf[base, sl]
                for j in range(1, SPAN):
                    acc = acc + buf[base + j, sl]
                out_v[b, sl] = acc * inv_span

        pltpu.async_copy(out_v, out_slice(s), sem_o)

    fire(0, rows0, sem0)

    def pair(i, _):
        s0 = 2 * i
        fire(s0 + 1, rows1, sem1)
        drain(rows0, sem0)

        @pl.when(i > 0)
        def _():
            pltpu.make_async_copy(out0, out_slice(0), sem_o0).wait()

        compute(s0, rows0, out0, sem_o0)

        s1 = 2 * i + 1

        @pl.when(i < n_macro // 2 - 1)
        def _():
            fire(s1 + 1, rows0, sem0)

        drain(rows1, sem1)

        @pl.when(i > 0)
        def _():
            pltpu.make_async_copy(out1, out_slice(0), sem_o1).wait()

        compute(s1, rows1, out1, sem_o1)
        return 0

    lax.fori_loop(0, n_macro // 2, pair, 0)
    pltpu.make_async_copy(out0, out_slice(0), sem_o0).wait()
    pltpu.make_async_copy(out1, out_slice(0), sem_o1).wait()


def kernel(words, table):
    batch, span = words.shape
    assert span == SPAN and table.shape[1] == DIM

    mesh = plsc.VectorSubcoreMesh(core_axis_name="c", subcore_axis_name="s")
    f = pl.kernel(
        _cbow_body,
        out_type=jax.ShapeDtypeStruct((batch, DIM), jnp.float32),
        mesh=mesh,
        scratch_types=[
            pltpu.VMEM((batch * SPAN // NUM_WORKERS,), jnp.int32),
            pltpu.VMEM((MACRO_B * SPAN, DIM), jnp.float32),
            pltpu.VMEM((MACRO_B * SPAN, DIM), jnp.float32),
            pltpu.VMEM((MACRO_B, DIM), jnp.float32),
            pltpu.VMEM((MACRO_B, DIM), jnp.float32),
            pltpu.SemaphoreType.DMA,
            pltpu.SemaphoreType.DMA,
            pltpu.SemaphoreType.DMA,
            pltpu.SemaphoreType.DMA,
        ],
    )
    return f(words.reshape(-1), table)


# speedup vs baseline: 1.0297x; 1.0297x over previous
import jax
import jax.numpy as jnp
from jax import lax
from jax.experimental import pallas as pl
from jax.experimental.pallas import tpu as pltpu
from jax.experimental.pallas import tpu_sc as plsc

DIM = 128
SPAN = 10
LANES = 16
NUM_WORKERS = 32
MACRO_B = 16
UNROLL = 4


def _cbow_body(words_hbm, table_hbm, out_hbm, idx_v, rows0, rows1, out0,
               out1, sem0, sem1, sem_o0, sem_o1):
    b_per_w = out_hbm.shape[0] // NUM_WORKERS
    n_macro = b_per_w // MACRO_B
    wid = lax.axis_index("s") * 2 + lax.axis_index("c")
    pltpu.sync_copy(words_hbm.at[pl.ds(wid * b_per_w, b_per_w)], idx_v)

    def fire(s, buf, sem):
        def one(g, _):
            base = s * MACRO_B + g * UNROLL
            for u in range(UNROLL):
                pltpu.async_copy(
                    table_hbm.at[idx_v.at[base + u]],
                    buf.at[pl.ds((g * UNROLL + u) * SPAN, SPAN)],
                    sem,
                )
            return 0

        lax.fori_loop(0, MACRO_B // UNROLL, one, 0)

    def drain(buf, sem):
        pltpu.make_async_copy(
            table_hbm.at[pl.ds(0, MACRO_B * SPAN)], buf, sem
        ).wait()

    def out_slice(s):
        return out_hbm.at[pl.ds(wid * b_per_w + s * MACRO_B, MACRO_B)]

    def compute(s, buf, out_v, sem_o):
        inv_span = jnp.float32(1.0 / SPAN)

        @plsc.parallel_loop(0, MACRO_B, unroll=4)
        def _(b):
            base = b * SPAN
            for g in range(DIM // LANES):
                sl = pl.ds(g * LANES, LANES)
                acc = buf[base, sl]
                for j in range(1, SPAN):
                    acc = acc + buf[base + j, sl]
                out_v[b, sl] = acc * inv_span

        pltpu.async_copy(out_v, out_slice(s), sem_o)

    fire(0, rows0, sem0)

    def pair(i, _):
        s0 = 2 * i
        fire(s0 + 1, rows1, sem1)
        drain(rows0, sem0)

        @pl.when(i > 0)
        def _():
            pltpu.make_async_copy(out0, out_slice(0), sem_o0).wait()

        compute(s0, rows0, out0, sem_o0)

        s1 = 2 * i + 1

        @pl.when(i < n_macro // 2 - 1)
        def _():
            fire(s1 + 1, rows0, sem0)

        drain(rows1, sem1)

        @pl.when(i > 0)
        def _():
            pltpu.make_async_copy(out1, out_slice(0), sem_o1).wait()

        compute(s1, rows1, out1, sem_o1)
        return 0

    lax.fori_loop(0, n_macro // 2, pair, 0)
    pltpu.make_async_copy(out0, out_slice(0), sem_o0).wait()
    pltpu.make_async_copy(out1, out_slice(0), sem_o1).wait()


def kernel(words, table):
    batch, span = words.shape
    assert span == SPAN and table.shape[1] == DIM

    mesh = plsc.VectorSubcoreMesh(core_axis_name="c", subcore_axis_name="s")
    f = pl.kernel(
        _cbow_body,
        out_type=jax.ShapeDtypeStruct((batch, DIM), jnp.float32),
        mesh=mesh,
        scratch_types=[
            pltpu.VMEM((batch // NUM_WORKERS, SPAN), jnp.int32),
            pltpu.VMEM((MACRO_B * SPAN, DIM), jnp.float32),
            pltpu.VMEM((MACRO_B * SPAN, DIM), jnp.float32),
            pltpu.VMEM((MACRO_B, DIM), jnp.float32),
            pltpu.VMEM((MACRO_B, DIM), jnp.float32),
            pltpu.SemaphoreType.DMA,
            pltpu.SemaphoreType.DMA,
            pltpu.SemaphoreType.DMA,
            pltpu.SemaphoreType.DMA,
        ],
    )
    return f(words, table)


# speedup vs baseline: 1.0377x; 1.0078x over previous
import jax
import jax.numpy as jnp
from jax import lax
from jax.experimental import pallas as pl
from jax.experimental.pallas import tpu as pltpu
from jax.experimental.pallas import tpu_sc as plsc

DIM = 128
SPAN = 10
LANES = 16
NUM_WORKERS = 32
MACRO_B = 16
UNROLL = 4


def _cbow_body(words_hbm, table_hbm, out_hbm, idx_v, rows0, rows1, out0,
               out1, sem0, sem1, sem_o0, sem_o1):
    b_per_w = out_hbm.shape[0] // NUM_WORKERS
    n_macro = b_per_w // MACRO_B
    wid = lax.axis_index("s") * 2 + lax.axis_index("c")
    pltpu.sync_copy(words_hbm.at[pl.ds(wid * b_per_w, b_per_w)], idx_v)

    def fire(s, buf, sem):
        def one(g, _):
            base = s * MACRO_B + g * UNROLL
            for u in range(UNROLL):
                pltpu.async_copy(
                    table_hbm.at[idx_v.at[base + u]],
                    buf.at[pl.ds((g * UNROLL + u) * SPAN, SPAN)],
                    sem,
                )
            return 0

        lax.fori_loop(0, MACRO_B // UNROLL, one, 0)

    def drain(buf, sem):
        pltpu.make_async_copy(
            table_hbm.at[pl.ds(0, MACRO_B * SPAN)], buf, sem
        ).wait()

    def out_slice(s):
        return out_hbm.at[pl.ds(wid * b_per_w + s * MACRO_B, MACRO_B)]

    def compute(s, buf, out_v, sem_o):
        inv_span = jnp.float32(1.0 / SPAN)

        @plsc.parallel_loop(0, MACRO_B, unroll=4)
        def _(b):
            base = b * SPAN
            for g in range(DIM // LANES):
                sl = pl.ds(g * LANES, LANES)
                terms = [buf[base + j, sl] for j in range(SPAN)]
                while len(terms) > 1:
                    terms = [
                        terms[t] + terms[t + 1] if t + 1 < len(terms)
                        else terms[t]
                        for t in range(0, len(terms), 2)
                    ]
                out_v[b, sl] = terms[0] * inv_span

        pltpu.async_copy(out_v, out_slice(s), sem_o)

    fire(0, rows0, sem0)

    def pair(i, _):
        s0 = 2 * i
        fire(s0 + 1, rows1, sem1)
        drain(rows0, sem0)

        @pl.when(i > 0)
        def _():
            pltpu.make_async_copy(out0, out_slice(0), sem_o0).wait()

        compute(s0, rows0, out0, sem_o0)

        s1 = 2 * i + 1

        @pl.when(i < n_macro // 2 - 1)
        def _():
            fire(s1 + 1, rows0, sem0)

        drain(rows1, sem1)

        @pl.when(i > 0)
        def _():
            pltpu.make_async_copy(out1, out_slice(0), sem_o1).wait()

        compute(s1, rows1, out1, sem_o1)
        return 0

    lax.fori_loop(0, n_macro // 2, pair, 0)
    pltpu.make_async_copy(out0, out_slice(0), sem_o0).wait()
    pltpu.make_async_copy(out1, out_slice(0), sem_o1).wait()


def kernel(words, table):
    batch, span = words.shape
    assert span == SPAN and table.shape[1] == DIM

    mesh = plsc.VectorSubcoreMesh(core_axis_name="c", subcore_axis_name="s")
    f = pl.kernel(
        _cbow_body,
        out_type=jax.ShapeDtypeStruct((batch, DIM), jnp.float32),
        mesh=mesh,
        scratch_types=[
            pltpu.VMEM((batch // NUM_WORKERS, SPAN), jnp.int32),
            pltpu.VMEM((MACRO_B * SPAN, DIM), jnp.float32),
            pltpu.VMEM((MACRO_B * SPAN, DIM), jnp.float32),
            pltpu.VMEM((MACRO_B, DIM), jnp.float32),
            pltpu.VMEM((MACRO_B, DIM), jnp.float32),
            pltpu.SemaphoreType.DMA,
            pltpu.SemaphoreType.DMA,
            pltpu.SemaphoreType.DMA,
            pltpu.SemaphoreType.DMA,
        ],
    )
    return f(words, table)
